# baseline (speedup 1.0000x reference)
import jax
import jax.numpy as jnp
from jax.experimental import pallas as pl
from jax.experimental.pallas import tpu as pltpu


def _fused_vo_body(x_ref, bv_ref, bo_ref, wv_ref, wo_ref, y_ref, v_acc):
    i = pl.program_id(0)
    h = wo_ref.shape[0]

    @pl.when(i == 0)
    def _():
        v_acc[...] = jnp.dot(x_ref[...], wv_ref[...],
                             preferred_element_type=jnp.float32) + bv_ref[...]
        y_ref[...] = jnp.dot(v_acc[:, :h], wo_ref[...],
                             preferred_element_type=jnp.float32) + bo_ref[...]

    @pl.when(i == 1)
    def _():
        y_ref[...] += jnp.dot(v_acc[:, h:], wo_ref[...],
                              preferred_element_type=jnp.float32)


def kernel(x, reset_mask, Wq, bq, Wk, bk, Wv, bv, Wo, bo):
    del reset_mask, Wq, bq, Wk, bk
    bs, d = x.shape
    d_wm = Wv.shape[1]
    h = d_wm // 2
    return pl.pallas_call(
        _fused_vo_body,
        grid=(2,),
        in_specs=[
            pl.BlockSpec((bs, d), lambda i: (0, 0)),
            pl.BlockSpec((1, d_wm), lambda i: (0, 0)),
            pl.BlockSpec((1, d), lambda i: (0, 0)),
            pl.BlockSpec((d, d_wm), lambda i: (0, 0)),
            pl.BlockSpec((h, d), lambda i: (i, 0)),
        ],
        out_specs=pl.BlockSpec((bs, d), lambda i: (0, 0)),
        out_shape=jax.ShapeDtypeStruct((bs, d), jnp.float32),
        scratch_shapes=[
            pltpu.VMEM((bs, d_wm), jnp.float32),
        ],
    )(x, bv.reshape(1, -1), bo.reshape(1, -1), Wv, Wo)
